# Initial kernel scaffold; baseline (speedup 1.0000x reference)
import jax
import jax.numpy as jnp
from jax.experimental import pallas as pl


def kernel(f1, f2, f3, table1, table2, W, b):
    raise NotImplementedError("write your pallas kernel here")


# speedup vs baseline: 4.1488x; 4.1488x over previous
import functools

import jax
import jax.numpy as jnp
from jax import lax
from jax.experimental import pallas as pl
from jax.experimental.pallas import tpu as pltpu
from jax.experimental.pallas import tpu_sc as plsc

L = 16


def _make_sc_kernel(B, D, NC, NS):
    NW = NC * NS
    SPW = B // NW
    GRP = 128
    G = SPW // GRP
    CH = D // L

    mesh = plsc.VectorSubcoreMesh(core_axis_name="c", subcore_axis_name="s")

    @functools.partial(
        pl.kernel,
        mesh=mesh,
        compiler_params=pltpu.CompilerParams(needs_layout_passes=False),
        out_type=jax.ShapeDtypeStruct((B,), jnp.float32),
        scratch_types=[
            pltpu.VMEM((G, GRP), jnp.int32),
            pltpu.VMEM((G, GRP), jnp.int32),
            pltpu.VMEM((G, GRP), jnp.int32),
            pltpu.VMEM((2, GRP, D), jnp.float32),
            pltpu.VMEM((2, GRP, D), jnp.float32),
            pltpu.VMEM((2, GRP, D), jnp.float32),
            pltpu.VMEM((3 * D,), jnp.float32),
            pltpu.VMEM((L,), jnp.float32),
            pltpu.VMEM((SPW,), jnp.float32),
            pltpu.VMEM((L * GRP,), jnp.float32),
            pltpu.SemaphoreType.DMA,
            pltpu.SemaphoreType.DMA,
        ],
    )
    def sc_kernel(idx1_hbm, idx2_hbm, idx3_hbm, t1_hbm, t2_hbm, w_hbm,
                  b_hbm, out_hbm, idx1_v, idx2_v, idx3_v, r1_v, r2_v, r3_v,
                  w_v, b_v, out_v, accs_v, sem0, sem1):
        wid = lax.axis_index("s") * NC + lax.axis_index("c")
        base = wid * SPW

        pltpu.sync_copy(idx1_hbm.at[wid], idx1_v)
        pltpu.sync_copy(idx2_hbm.at[wid], idx2_v)
        pltpu.sync_copy(idx3_hbm.at[wid], idx3_v)
        pltpu.sync_copy(w_hbm, w_v)
        pltpu.sync_copy(b_hbm, b_v)

        wv = [w_v[pl.ds(16 * k, L)] for k in range(3 * CH)]
        bias_vec = b_v[pl.ds(0, L)]
        iota_scaled = lax.iota(jnp.int32, L) * GRP

        sems = (sem0, sem1)

        def start_group(g):
            buf = g % 2
            sem = sems[buf]
            return (
                pltpu.async_copy(t1_hbm.at[idx1_v.at[g]], r1_v.at[buf], sem),
                pltpu.async_copy(t1_hbm.at[idx2_v.at[g]], r2_v.at[buf], sem),
                pltpu.async_copy(t2_hbm.at[idx3_v.at[g]], r3_v.at[buf], sem),
            )

        pending = start_group(0)
        for g in range(G):
            nxt = start_group(g + 1) if g + 1 < G else ()
            for h in pending:
                h.wait()
            pending = nxt

            buf = g % 2

            def body(i, carry):
                acc = r1_v[buf, i, pl.ds(0, L)] * wv[0]
                for c in range(1, CH):
                    acc += r1_v[buf, i, pl.ds(16 * c, L)] * wv[c]
                for c in range(CH):
                    acc += r2_v[buf, i, pl.ds(16 * c, L)] * wv[CH + c]
                for c in range(CH):
                    acc += r3_v[buf, i, pl.ds(16 * c, L)] * wv[2 * CH + c]
                plsc.store_scatter(accs_v, [iota_scaled + i], acc)
                return carry

            lax.fori_loop(0, GRP, body, 0, unroll=False)

            for v in range(GRP // L):
                tot = accs_v[pl.ds(v * L, L)]
                for k in range(1, L):
                    tot += accs_v[pl.ds(k * GRP + v * L, L)]
                out_v[pl.ds(g * GRP + v * L, L)] = tot + bias_vec

        pltpu.sync_copy(out_v, out_hbm.at[pl.ds(base, SPW)])

    return sc_kernel


def kernel(f1, f2, f3, table1, table2, W, b):
    B = f1.shape[0]
    D = table1.shape[1]
    info = plsc.get_sparse_core_info()
    NC, NS = info.num_cores, info.num_subcores
    NW = NC * NS
    GRP = 128

    idx1 = f1.astype(jnp.int32).reshape(NW, B // (NW * GRP), GRP)
    idx2 = f2.astype(jnp.int32).reshape(NW, B // (NW * GRP), GRP)
    idx3 = f3.astype(jnp.int32).reshape(NW, B // (NW * GRP), GRP)
    w = W.reshape(3 * D)
    bpad = jnp.broadcast_to(b.astype(jnp.float32), (L,))

    sc = _make_sc_kernel(B, D, NC, NS)
    out = sc(idx1, idx2, idx3, table1, table2, w, bpad)
    return out.reshape(B, 1)
